# baseline (speedup 1.0000x reference)
import functools

import jax
import jax.numpy as jnp
from jax import lax
from jax.experimental import pallas as pl
from jax.experimental.pallas import tpu as pltpu
from jax.experimental.pallas import tpu_sc as plsc

NB_TOKENS = 1000000
DIM = 64
ROWS, COLS = 16384, 26
B = ROWS * COLS
NC, NS = 2, 16
NW = NC * NS
BPW = B // NW
CHUNK = 128
NCHUNK = BPW // CHUNK
NBUF = 4
LAG = 2

TBLK = 128
NFULL = NB_TOKENS // TBLK
TREM = NFULL % NW
TPW = NFULL // NW
TAIL_I0 = NFULL * TBLK
VOUT_W = 130

_mesh = plsc.VectorSubcoreMesh(core_axis_name="c", subcore_axis_name="s")


@functools.partial(
    pl.kernel,
    mesh=_mesh,
    out_type=jax.ShapeDtypeStruct((NB_TOKENS // 2, 2 * DIM), jnp.float32),
    scratch_types=[
        pltpu.VMEM((2, DIM, TBLK), jnp.float32),
        pltpu.VMEM((2, DIM, VOUT_W), jnp.float32),
        pltpu.SemaphoreType.DMA,
        pltpu.SemaphoreType.DMA,
    ],
    compiler_params=pltpu.CompilerParams(
        use_tc_tiling_on_sc=True, needs_layout_passes=False),
)
def _table_relayout(wt_hbm, tail_hbm, out_hbm, vin, vout, isem, osem):
    wid = lax.axis_index("s") * NC + lax.axis_index("c")
    nblk = jnp.where(wid < TREM, TPW + 1, TPW)

    def blk(t):
        return wid + NW * t

    def start_in(t, sb):
        pltpu.async_copy(
            wt_hbm.at[:, pl.ds(blk(t) * TBLK, TBLK)], vin.at[sb], isem)

    def wait_in(t, sb):
        pltpu.make_async_copy(
            wt_hbm.at[:, pl.ds(blk(t) * TBLK, TBLK)], vin.at[sb], isem).wait()

    def start_out(t, sb):
        pltpu.async_copy(
            vout.at[sb, :, pl.ds(0, TBLK)],
            out_hbm.at[pl.ds(blk(t) * (TBLK // 2), TBLK // 2), :], osem)

    def wait_out(t, sb):
        pltpu.make_async_copy(
            vout.at[sb, :, pl.ds(0, TBLK)],
            out_hbm.at[pl.ds(blk(t) * (TBLK // 2), TBLK // 2), :], osem).wait()

    iota = lax.iota(jnp.int32, 16)
    row_half = iota >> 1
    colbase = (iota & 1) * DIM
    rows8 = [(ii0 >> 1) + row_half for ii0 in range(0, TBLK, 16)]

    def transpose_block(sb):
        for d in range(DIM):
            cols = colbase + d
            for g in range(TBLK // 16):
                v = vin[sb, d, pl.ds(g * 16, 16)]
                plsc.store_scatter(vout.at[sb], [rows8[g], cols], v)

    start_in(0, 0)

    def body(t2, carry):
        for sb in range(2):
            t = 2 * t2 + sb

            @pl.when(t < nblk)
            def _process():
                @pl.when(t + 1 < nblk)
                def _prefetch():
                    start_in(t + 1, (sb + 1) % 2)

                wait_in(t, sb)

                @pl.when(t >= 2)
                def _drain():
                    wait_out(t - 2, sb)

                transpose_block(sb)
                start_out(t, sb)
        return carry

    lax.fori_loop(0, (TPW + 2) // 2, body, 0)
    wait_out(nblk - 2, (nblk - 2) % 2)
    wait_out(nblk - 1, (nblk - 1) % 2)

    @pl.when(wid == NW - 1)
    def _tail():
        pltpu.sync_copy(tail_hbm, vin.at[0, pl.ds(0, 32), :])
        pltpu.sync_copy(
            vin.at[0, pl.ds(0, 32), :],
            out_hbm.at[pl.ds(TAIL_I0 // 2, 32), :])


@functools.partial(
    pl.kernel,
    mesh=_mesh,
    out_type=jax.ShapeDtypeStruct((B, DIM), jnp.float32),
    scratch_types=[
        pltpu.VMEM((NCHUNK, CHUNK), jnp.int32),
        pltpu.VMEM((NBUF, CHUNK, DIM), jnp.float32),
        pltpu.SemaphoreType.DMA,
        pltpu.SemaphoreType.DMA,
    ],
    compiler_params=pltpu.CompilerParams(use_tc_tiling_on_sc=False),
)
def _emb_lookup(idx_hbm, table_hbm, out_hbm, idx_v, rows_v, gsem, ssem):
    wid = lax.axis_index("s") * NC + lax.axis_index("c")
    pltpu.sync_copy(idx_hbm.at[wid], idx_v)
    base = wid * BPW

    def start_gather(j):
        pltpu.async_copy(table_hbm.at[idx_v.at[j]], rows_v.at[j % NBUF], gsem)

    def wait_gather(j):
        pltpu.make_async_copy(
            table_hbm.at[idx_v.at[j]], rows_v.at[j % NBUF], gsem).wait()

    def start_scatter(j):
        pltpu.async_copy(
            rows_v.at[j % NBUF], out_hbm.at[pl.ds(base + j * CHUNK, CHUNK)],
            ssem)

    def wait_scatter(j):
        pltpu.make_async_copy(
            rows_v.at[j % NBUF], out_hbm.at[pl.ds(base + j * CHUNK, CHUNK)],
            ssem).wait()

    for j in range(LAG):
        start_gather(j)
    for j in range(LAG):
        start_gather(j + LAG)
        wait_gather(j)
        start_scatter(j)

    def body(j, carry):
        wait_scatter(j - LAG)
        start_gather(j + LAG)
        wait_gather(j)
        start_scatter(j)
        return carry

    lax.fori_loop(LAG, NCHUNK - LAG, body, 0)

    for j in range(NCHUNK - LAG, NCHUNK):
        wait_scatter(j - LAG)
        wait_gather(j)
        start_scatter(j)
    for j in range(NCHUNK - LAG, NCHUNK):
        wait_scatter(j)


def kernel(x, weight):
    wt = jnp.swapaxes(weight, 0, 1)
    tail = lax.slice(weight, (TAIL_I0, 0), (NB_TOKENS, DIM)).reshape(32, 128)
    table = _table_relayout(wt, tail).reshape(NB_TOKENS, DIM)
    idx = jnp.swapaxes(x, 0, 1).reshape(NW, NCHUNK, CHUNK).astype(jnp.int32)
    out = _emb_lookup(idx, table)
    return out.reshape(COLS, ROWS, DIM).transpose(1, 0, 2)


# speedup vs baseline: 1.0104x; 1.0104x over previous
import functools

import jax
import jax.numpy as jnp
from jax import lax
from jax.experimental import pallas as pl
from jax.experimental.pallas import tpu as pltpu
from jax.experimental.pallas import tpu_sc as plsc

NB_TOKENS = 1000000
DIM = 64
ROWS, COLS = 16384, 26
B = ROWS * COLS
NC, NS = 2, 16
NW = NC * NS
BPW = B // NW
CHUNK = 128
NCHUNK = BPW // CHUNK
NBUF = 4
LAG = 2

TBLK = 128
NFULL = NB_TOKENS // TBLK
TREM = NFULL % NW
TPW = NFULL // NW
TAIL_I0 = NFULL * TBLK
VOUT_W = 130

_mesh = plsc.VectorSubcoreMesh(core_axis_name="c", subcore_axis_name="s")


@functools.partial(
    pl.kernel,
    mesh=_mesh,
    out_type=jax.ShapeDtypeStruct((NB_TOKENS // 2, 2 * DIM), jnp.float32),
    scratch_types=[
        pltpu.VMEM((2, DIM, TBLK), jnp.float32),
        pltpu.VMEM((2, DIM, VOUT_W), jnp.float32),
        pltpu.SemaphoreType.DMA,
        pltpu.SemaphoreType.DMA,
    ],
    compiler_params=pltpu.CompilerParams(
        use_tc_tiling_on_sc=True, needs_layout_passes=False),
)
def _table_relayout(wt_hbm, tail_hbm, out_hbm, vin, vout, isem, osem):
    wid = lax.axis_index("s") * NC + lax.axis_index("c")
    nblk = jnp.where(wid < TREM, TPW + 1, TPW)

    def blk(t):
        return wid + NW * t

    def start_in(t, sb):
        pltpu.async_copy(
            wt_hbm.at[:, pl.ds(blk(t) * TBLK, TBLK)], vin.at[sb], isem)

    def wait_in(t, sb):
        pltpu.make_async_copy(
            wt_hbm.at[:, pl.ds(blk(t) * TBLK, TBLK)], vin.at[sb], isem).wait()

    def start_out(t, sb):
        pltpu.async_copy(
            vout.at[sb, :, pl.ds(0, TBLK)],
            out_hbm.at[pl.ds(blk(t) * (TBLK // 2), TBLK // 2), :], osem)

    def wait_out(t, sb):
        pltpu.make_async_copy(
            vout.at[sb, :, pl.ds(0, TBLK)],
            out_hbm.at[pl.ds(blk(t) * (TBLK // 2), TBLK // 2), :], osem).wait()

    iota = lax.iota(jnp.int32, 16)
    row_half = iota >> 1
    colbase = (iota & 1) * DIM
    rows8 = [(ii0 >> 1) + row_half for ii0 in range(0, TBLK, 16)]

    def transpose_block(sb):
        for d in range(DIM):
            cols = colbase + d
            vs = [vin[sb, d, pl.ds(g * 16, 16)] for g in range(TBLK // 16)]
            for g in range(TBLK // 16):
                plsc.store_scatter(vout.at[sb], [rows8[g], cols], vs[g])

    start_in(0, 0)

    def body(t2, carry):
        for sb in range(2):
            t = 2 * t2 + sb

            @pl.when(t < nblk)
            def _process():
                @pl.when(t + 1 < nblk)
                def _prefetch():
                    start_in(t + 1, (sb + 1) % 2)

                wait_in(t, sb)

                @pl.when(t >= 2)
                def _drain():
                    wait_out(t - 2, sb)

                transpose_block(sb)
                start_out(t, sb)
        return carry

    lax.fori_loop(0, (TPW + 2) // 2, body, 0)
    wait_out(nblk - 2, (nblk - 2) % 2)
    wait_out(nblk - 1, (nblk - 1) % 2)

    @pl.when(wid == NW - 1)
    def _tail():
        pltpu.sync_copy(tail_hbm, vin.at[0, pl.ds(0, 32), :])
        pltpu.sync_copy(
            vin.at[0, pl.ds(0, 32), :],
            out_hbm.at[pl.ds(TAIL_I0 // 2, 32), :])


@functools.partial(
    pl.kernel,
    mesh=_mesh,
    out_type=jax.ShapeDtypeStruct((B, DIM), jnp.float32),
    scratch_types=[
        pltpu.VMEM((NCHUNK, CHUNK), jnp.int32),
        pltpu.VMEM((NBUF, CHUNK, DIM), jnp.float32),
        pltpu.SemaphoreType.DMA,
        pltpu.SemaphoreType.DMA,
    ],
    compiler_params=pltpu.CompilerParams(use_tc_tiling_on_sc=False),
)
def _emb_lookup(idx_hbm, table_hbm, out_hbm, idx_v, rows_v, gsem, ssem):
    wid = lax.axis_index("s") * NC + lax.axis_index("c")
    pltpu.sync_copy(idx_hbm.at[wid], idx_v)
    base = wid * BPW

    def start_gather(j):
        pltpu.async_copy(table_hbm.at[idx_v.at[j]], rows_v.at[j % NBUF], gsem)

    def wait_gather(j):
        pltpu.make_async_copy(
            table_hbm.at[idx_v.at[j]], rows_v.at[j % NBUF], gsem).wait()

    def start_scatter(j):
        pltpu.async_copy(
            rows_v.at[j % NBUF], out_hbm.at[pl.ds(base + j * CHUNK, CHUNK)],
            ssem)

    def wait_scatter(j):
        pltpu.make_async_copy(
            rows_v.at[j % NBUF], out_hbm.at[pl.ds(base + j * CHUNK, CHUNK)],
            ssem).wait()

    for j in range(LAG):
        start_gather(j)
    for j in range(LAG):
        start_gather(j + LAG)
        wait_gather(j)
        start_scatter(j)

    def body(j, carry):
        wait_scatter(j - LAG)
        start_gather(j + LAG)
        wait_gather(j)
        start_scatter(j)
        return carry

    lax.fori_loop(LAG, NCHUNK - LAG, body, 0)

    for j in range(NCHUNK - LAG, NCHUNK):
        wait_scatter(j - LAG)
        wait_gather(j)
        start_scatter(j)
    for j in range(NCHUNK - LAG, NCHUNK):
        wait_scatter(j)


def kernel(x, weight):
    wt = jnp.swapaxes(weight, 0, 1)
    tail = lax.slice(weight, (TAIL_I0, 0), (NB_TOKENS, DIM)).reshape(32, 128)
    table = _table_relayout(wt, tail).reshape(NB_TOKENS, DIM)
    idx = jnp.swapaxes(x, 0, 1).reshape(NW, NCHUNK, CHUNK).astype(jnp.int32)
    out = _emb_lookup(idx, table)
    return out.reshape(COLS, ROWS, DIM).transpose(1, 0, 2)
